# baseline (speedup 1.0000x reference)
import functools

import jax
import jax.numpy as jnp
from jax import lax
from jax.experimental import pallas as pl
from jax.experimental.pallas import tpu as pltpu
from jax.experimental.pallas import tpu_sc as plsc

NC = 2
NS = 16
K = 64


def _sc_aggregate(x, sr):
    n, d = x.shape
    ch = sr.shape[1]
    k = sr.shape[3]
    rpt = ((n // NS) + 7) // 8 * 8
    rlast = n - (NS - 1) * rpt
    na = n + 16

    mesh = plsc.VectorSubcoreMesh(core_axis_name="c", subcore_axis_name="s")

    @functools.partial(
        pl.kernel,
        mesh=mesh,
        out_type=jax.ShapeDtypeStruct((NC, n, d), jnp.float32),
        scratch_types=[
            [pltpu.VMEM((2, k), jnp.int32) for _ in range(8)],
            [pltpu.VMEM((k, d), jnp.float32) for _ in range(4)],
            pltpu.VMEM_SHARED((na, d), jnp.float32),
            [pltpu.SemaphoreType.DMA for _ in range(8)],
            [pltpu.SemaphoreType.DMA for _ in range(4)],
            [pltpu.SemaphoreType.DMA for _ in range(4)],
            pltpu.SemaphoreType.DMA,
        ],
    )
    def run(x_hbm, sr_hbm, out_hbm, idx, rows, acc, si, sg, ss, s_init):
        c = lax.axis_index("c")
        s = lax.axis_index("s")
        rbase = s * rpt
        wid = c * NS + s

        @pl.when(s < NS - 1)
        def _():
            pltpu.async_copy(x_hbm.at[pl.ds(rbase, rpt)],
                             acc.at[pl.ds(rbase, rpt)], s_init)

        @pl.when(s == NS - 1)
        def _():
            pltpu.async_copy(x_hbm.at[pl.ds(rbase, rlast)],
                             acc.at[pl.ds(rbase, rlast)], s_init)

        for t in range(4):
            pltpu.async_copy(sr_hbm.at[wid, t], idx[t], si[t])

        @pl.when(s < NS - 1)
        def _():
            pltpu.make_async_copy(x_hbm.at[pl.ds(rbase, rpt)],
                                  acc.at[pl.ds(rbase, rpt)], s_init).wait()

        @pl.when(s == NS - 1)
        def _():
            pltpu.make_async_copy(x_hbm.at[pl.ds(rbase, rlast)],
                                  acc.at[pl.ds(rbase, rlast)], s_init).wait()

        plsc.subcore_barrier()

        def scat_wait(j8, j4):
            pltpu.make_async_copy(rows[j4], acc.at[idx[j8].at[1]],
                                  ss[j4]).wait()

        def chunk_step(i, t8):
            t4 = t8 % 4

            @pl.when(i >= 4)
            def _():
                scat_wait(t8, t4)

            pltpu.make_async_copy(sr_hbm.at[wid, i], idx[t8], si[t8]).wait()
            pltpu.async_copy(x_hbm.at[idx[t8].at[0]], rows[t4], sg[t4])

            @pl.when(i + 4 < ch)
            def _():
                pltpu.async_copy(sr_hbm.at[wid, i + 4], idx[(t8 + 4) % 8],
                                 si[(t8 + 4) % 8])

            @pl.when(i >= 2)
            def _():
                p8, p4 = (t8 - 2) % 8, (t8 - 2) % 4
                pltpu.make_async_copy(x_hbm.at[idx[p8].at[0]], rows[p4],
                                      sg[p4]).wait()
                pltpu.async_copy(rows[p4], acc.at[idx[p8].at[1]], ss[p4],
                                 add=True)

        def body(j, carry):
            for t in range(8):
                chunk_step(8 * j + t, t)
            return carry

        lax.fori_loop(0, ch // 8, body, 0)

        for j in (ch - 2, ch - 1):
            pltpu.make_async_copy(x_hbm.at[idx[j % 8].at[0]], rows[j % 4],
                                  sg[j % 4]).wait()
            pltpu.async_copy(rows[j % 4], acc.at[idx[j % 8].at[1]],
                             ss[j % 4], add=True)
        for j in range(ch - 4, ch):
            scat_wait(j % 8, j % 4)
        plsc.subcore_barrier()

        @pl.when(s < NS - 1)
        def _():
            pltpu.sync_copy(acc.at[pl.ds(rbase, rpt)],
                            out_hbm.at[c, pl.ds(rbase, rpt)])

        @pl.when(s == NS - 1)
        def _():
            pltpu.sync_copy(acc.at[pl.ds(rbase, rlast)],
                            out_hbm.at[c, pl.ds(rbase, rlast)])

    return run(x, sr)


def _dense0(y_ref, nodes_ref, w0_ref, b0_ref, w1a_ref, w1b_ref, b1_ref,
            p0_ref, r0_ref):
    agg0 = y_ref[0] + y_ref[1] - nodes_ref[...]
    h0 = jnp.maximum(agg0 @ w0_ref[...] + b0_ref[...], 0.0)
    p0 = h0 @ w1a_ref[...]
    p0_ref[...] = p0
    r0_ref[...] = agg0 @ w1b_ref[...] + b1_ref[...] - p0


def _dense1(z_ref, r0_ref, mask_ref, starts_ref, ends_ref, wg_ref, bg_ref,
            out_ref):
    g = out_ref.shape[0]
    n = r0_ref.shape[0]
    h1 = jnp.maximum(z_ref[0] + z_ref[1] + r0_ref[...], 0.0)
    col = lax.broadcasted_iota(jnp.int32, (g, n), 1)
    sel = (col >= starts_ref[...]) & (col < ends_ref[...])
    onehot = jnp.where(sel, mask_ref[...], 0.0)
    hg = jnp.dot(onehot, h1, preferred_element_type=jnp.float32)
    out_ref[...] = hg @ wg_ref[...] + bg_ref[...]


def kernel(nodes, senders, receivers, n_node, is_root_mask,
           W0, b0, W1, b1, Wg, bg):
    n, d = nodes.shape
    g = n_node.shape[0]
    out_d = Wg.shape[1]

    e = senders.shape[0]
    ch = -(-e // (NC * NS * K * 8)) * 8
    pad = NC * NS * ch * K - e
    sp = jnp.concatenate([senders, jnp.zeros((pad,), senders.dtype)])
    rp = jnp.concatenate([receivers, jnp.full((pad,), n, receivers.dtype)])
    sr = jnp.stack([sp.reshape(NC * NS, ch, K),
                    rp.reshape(NC * NS, ch, K)], axis=2)
    w1a = W1[:d]
    w1b = W1[d:]
    maskf = is_root_mask.astype(jnp.float32).reshape(1, n)
    ends = jnp.cumsum(n_node).reshape(g, 1)
    starts = ends - n_node.reshape(g, 1)

    y = _sc_aggregate(nodes, sr)

    p0, r0 = pl.pallas_call(
        _dense0,
        out_shape=(jax.ShapeDtypeStruct((n, d), jnp.float32),
                   jax.ShapeDtypeStruct((n, d), jnp.float32)),
    )(y, nodes, W0, b0.reshape(1, -1), w1a, w1b, b1.reshape(1, -1))

    z = _sc_aggregate(p0, sr)

    out = pl.pallas_call(
        _dense1,
        out_shape=jax.ShapeDtypeStruct((g, out_d), jnp.float32),
    )(z, r0, maskf, starts, ends, Wg, bg.reshape(1, -1))
    return out


# speedup vs baseline: 4.2610x; 4.2610x over previous
import functools

import jax
import jax.numpy as jnp
from jax import lax
from jax.experimental import pallas as pl
from jax.experimental.pallas import tpu as pltpu
from jax.experimental.pallas import tpu_sc as plsc

NC = 2
NS = 16
K = 125


def _sc_aggregate(x, sr):
    n, d = x.shape
    ch = sr.shape[1]
    k = sr.shape[3]
    rpt = ((n // NS) + 7) // 8 * 8
    rlast = n - (NS - 1) * rpt
    na = n + 16

    mesh = plsc.VectorSubcoreMesh(core_axis_name="c", subcore_axis_name="s")

    @functools.partial(
        pl.kernel,
        mesh=mesh,
        out_type=jax.ShapeDtypeStruct((NC, n, d), jnp.float32),
        scratch_types=[
            [pltpu.VMEM((2, k), jnp.int32) for _ in range(4)],
            [pltpu.VMEM((k, d), jnp.float32) for _ in range(2)],
            pltpu.VMEM_SHARED((na, d), jnp.float32),
            [pltpu.SemaphoreType.DMA for _ in range(4)],
            [pltpu.SemaphoreType.DMA for _ in range(2)],
            [pltpu.SemaphoreType.DMA for _ in range(2)],
            pltpu.SemaphoreType.DMA,
        ],
    )
    def run(x_hbm, sr_hbm, out_hbm, idx, rows, acc, si, sg, ss, s_init):
        c = lax.axis_index("c")
        s = lax.axis_index("s")
        rbase = s * rpt
        wid = c * NS + s

        @pl.when(s < NS - 1)
        def _():
            pltpu.async_copy(x_hbm.at[pl.ds(rbase, rpt)],
                             acc.at[pl.ds(rbase, rpt)], s_init)

        @pl.when(s == NS - 1)
        def _():
            pltpu.async_copy(x_hbm.at[pl.ds(rbase, rlast)],
                             acc.at[pl.ds(rbase, rlast)], s_init)

        for t in range(2):
            pltpu.async_copy(sr_hbm.at[wid, t], idx[t], si[t])

        @pl.when(s < NS - 1)
        def _():
            pltpu.make_async_copy(x_hbm.at[pl.ds(rbase, rpt)],
                                  acc.at[pl.ds(rbase, rpt)], s_init).wait()

        @pl.when(s == NS - 1)
        def _():
            pltpu.make_async_copy(x_hbm.at[pl.ds(rbase, rlast)],
                                  acc.at[pl.ds(rbase, rlast)], s_init).wait()

        plsc.subcore_barrier()

        def chunk_step(i, t):
            ia, ip, inx = idx[t % 4], idx[(t - 1) % 4], idx[(t + 2) % 4]
            sia, sin = si[t % 4], si[(t + 2) % 4]
            ra, rp = rows[t % 2], rows[(t - 1) % 2]
            sga, sgp = sg[t % 2], sg[(t - 1) % 2]
            ssa, ssp = ss[t % 2], ss[(t - 1) % 2]
            pltpu.make_async_copy(sr_hbm.at[wid, i], ia, sia).wait()

            @pl.when(i >= 2)
            def _():
                pltpu.make_async_copy(ra, acc.at[ia.at[1]], ssa).wait()

            pltpu.async_copy(x_hbm.at[ia.at[0]], ra, sga)

            @pl.when(i + 2 < ch)
            def _():
                pltpu.async_copy(sr_hbm.at[wid, i + 2], inx, sin)

            @pl.when(i > 0)
            def _():
                pltpu.make_async_copy(x_hbm.at[ip.at[0]], rp, sgp).wait()
                pltpu.async_copy(rp, acc.at[ip.at[1]], ssp, add=True)

        def body(j, carry):
            for t in range(4):
                chunk_step(4 * j + t, t)
            return carry

        lax.fori_loop(0, ch // 4, body, 0)
        il, rl = idx[(ch - 1) % 4], rows[(ch - 1) % 2]
        sgl, ssl = sg[(ch - 1) % 2], ss[(ch - 1) % 2]
        rq, ssq = rows[ch % 2], ss[ch % 2]
        pltpu.make_async_copy(x_hbm.at[il.at[0]], rl, sgl).wait()
        pltpu.async_copy(rl, acc.at[il.at[1]], ssl, add=True)
        pltpu.make_async_copy(rq, acc.at[il.at[1]], ssq).wait()
        pltpu.make_async_copy(rl, acc.at[il.at[1]], ssl).wait()
        plsc.subcore_barrier()

        @pl.when(s < NS - 1)
        def _():
            pltpu.sync_copy(acc.at[pl.ds(rbase, rpt)],
                            out_hbm.at[c, pl.ds(rbase, rpt)])

        @pl.when(s == NS - 1)
        def _():
            pltpu.sync_copy(acc.at[pl.ds(rbase, rlast)],
                            out_hbm.at[c, pl.ds(rbase, rlast)])

    return run(x, sr)


def _dense0(y_ref, nodes_ref, w0_ref, b0_ref, w1a_ref, w1b_ref, b1_ref,
            p0_ref, r0_ref):
    agg0 = y_ref[0] + y_ref[1] - nodes_ref[...]
    h0 = jnp.maximum(agg0 @ w0_ref[...] + b0_ref[...], 0.0)
    p0 = h0 @ w1a_ref[...]
    p0_ref[...] = p0
    r0_ref[...] = agg0 @ w1b_ref[...] + b1_ref[...] - p0


def _dense1(z_ref, r0_ref, mask_ref, starts_ref, ends_ref, wg_ref, bg_ref,
            out_ref):
    g = out_ref.shape[0]
    n = r0_ref.shape[0]
    h1 = jnp.maximum(z_ref[0] + z_ref[1] + r0_ref[...], 0.0)
    col = lax.broadcasted_iota(jnp.int32, (g, n), 1)
    sel = (col >= starts_ref[...]) & (col < ends_ref[...])
    onehot = jnp.where(sel, mask_ref[...], 0.0)
    hg = jnp.dot(onehot, h1, preferred_element_type=jnp.float32)
    out_ref[...] = hg @ wg_ref[...] + bg_ref[...]


def kernel(nodes, senders, receivers, n_node, is_root_mask,
           W0, b0, W1, b1, Wg, bg):
    n, d = nodes.shape
    g = n_node.shape[0]
    out_d = Wg.shape[1]

    e = senders.shape[0]
    ch = -(-e // (NC * NS * K * 8)) * 8
    pad = NC * NS * ch * K - e
    sp = jnp.concatenate([senders, jnp.zeros((pad,), senders.dtype)])
    rp = jnp.concatenate([receivers, jnp.full((pad,), n, receivers.dtype)])
    sr = jnp.stack([sp.reshape(NC * NS, ch, K),
                    rp.reshape(NC * NS, ch, K)], axis=2)
    w1a = W1[:d]
    w1b = W1[d:]
    maskf = is_root_mask.astype(jnp.float32).reshape(1, n)
    ends = jnp.cumsum(n_node).reshape(g, 1)
    starts = ends - n_node.reshape(g, 1)

    y = _sc_aggregate(nodes, sr)

    p0, r0 = pl.pallas_call(
        _dense0,
        out_shape=(jax.ShapeDtypeStruct((n, d), jnp.float32),
                   jax.ShapeDtypeStruct((n, d), jnp.float32)),
    )(y, nodes, W0, b0.reshape(1, -1), w1a, w1b, b1.reshape(1, -1))

    z = _sc_aggregate(p0, sr)

    out = pl.pallas_call(
        _dense1,
        out_shape=jax.ShapeDtypeStruct((g, out_d), jnp.float32),
    )(z, r0, maskf, starts, ends, Wg, bg.reshape(1, -1))
    return out
